# Initial kernel scaffold; baseline (speedup 1.0000x reference)
import jax
import jax.numpy as jnp
from jax.experimental import pallas as pl


def kernel(g, h, W, V, lin_w, lin_b):
    raise NotImplementedError("write your pallas kernel here")


# speedup vs baseline: 1.7334x; 1.7334x over previous
import jax
import jax.numpy as jnp
from jax.experimental import pallas as pl

_BB = 256


def _gin_block(h_ref, w_ref, v_ref, lw_ref, lb_ref, out_ref):
    hb = h_ref[:]
    bb, n, d = hb.shape
    feat = jnp.dot(
        hb.reshape(bb * n, d), w_ref[:], preferred_element_type=jnp.float32
    )
    feat = feat.reshape(bb, n, -1)
    k = n
    while k > 1:
        k //= 2
        feat = feat[:, :k, :] * feat[:, k:, :]
    pooled = feat[:, 0, :]
    m = jnp.dot(lw_ref[:], v_ref[:], preferred_element_type=jnp.float32)
    out_ref[:] = (
        jnp.dot(pooled, m.T, preferred_element_type=jnp.float32) + lb_ref[:]
    )


def kernel(g, h, W, V, lin_w, lin_b):
    del g
    B, N, D = h.shape
    O, H = lin_w.shape
    R = W.shape[1]
    lb2 = lin_b.reshape(1, O)
    grid = (B // _BB,)
    return pl.pallas_call(
        _gin_block,
        grid=grid,
        in_specs=[
            pl.BlockSpec((_BB, N, D), lambda i: (i, 0, 0)),
            pl.BlockSpec((D, R), lambda i: (0, 0)),
            pl.BlockSpec((H, R), lambda i: (0, 0)),
            pl.BlockSpec((O, H), lambda i: (0, 0)),
            pl.BlockSpec((1, O), lambda i: (0, 0)),
        ],
        out_specs=pl.BlockSpec((_BB, O), lambda i: (i, 0)),
        out_shape=jax.ShapeDtypeStruct((B, O), jnp.float32),
    )(h, W, V, lin_w, lb2)
